# baseline (speedup 1.0000x reference)
import functools

import jax
import jax.numpy as jnp
from jax import lax
from jax.experimental import pallas as pl
from jax.experimental.pallas import tpu as pltpu
from jax.experimental.pallas import tpu_sc as plsc

N = 10000
E = 320000
D_IN = 128
H = 16

NC = 2
NS = 16
NW = NC * NS
CHUNK = 128
NB_TOT = E // CHUNK
NB0 = NB_TOT // NW
NX = NB_TOT - NW * NB0

_f32 = jnp.float32


def _make_edge_pass(D):
    mesh = plsc.VectorSubcoreMesh(core_axis_name="c", subcore_axis_name="s")
    rows_per_sub = N // NS

    @functools.partial(
        pl.kernel,
        mesh=mesh,
        out_type=jax.ShapeDtypeStruct((NC, N, D), _f32),
        compiler_params=pltpu.CompilerParams(use_tc_tiling_on_sc=False),
        scratch_types=[
            pltpu.VMEM((NB0, CHUNK), jnp.int32),
            pltpu.VMEM((NB0, CHUNK), jnp.int32),
            pltpu.VMEM((1, CHUNK), jnp.int32),
            pltpu.VMEM((1, CHUNK), jnp.int32),
            pltpu.VMEM((CHUNK, D), _f32),
            pltpu.VMEM((CHUNK, D), _f32),
            pltpu.VMEM_SHARED((N, D), _f32),
            pltpu.SemaphoreType.DMA,
            pltpu.SemaphoreType.DMA,
            pltpu.SemaphoreType.DMA,
            pltpu.SemaphoreType.DMA,
        ],
    )
    def edge_pass(table_hbm, edge_hbm, zero_hbm, out_hbm,
                  src_v, dst_v, src_x, dst_x, rows0, rows1, acc_sh,
                  sem_g0, sem_g1, sem_s0, sem_s1):
        c = lax.axis_index("c")
        s = lax.axis_index("s")
        w = c * NS + s
        r0 = s * rows_per_sub
        pltpu.sync_copy(zero_hbm.at[pl.ds(r0, rows_per_sub)],
                        acc_sh.at[pl.ds(r0, rows_per_sub)])
        pltpu.sync_copy(edge_hbm.at[0, pl.ds(w * NB0, NB0)], src_v)
        pltpu.sync_copy(edge_hbm.at[1, pl.ds(w * NB0, NB0)], dst_v)

        @pl.when(w < NX)
        def _():
            pltpu.sync_copy(edge_hbm.at[0, pl.ds(NW * NB0 + w, 1)], src_x)
            pltpu.sync_copy(edge_hbm.at[1, pl.ds(NW * NB0 + w, 1)], dst_x)

        pltpu.async_copy(table_hbm.at[src_v.at[0]], rows0, sem_g0)
        plsc.subcore_barrier()

        @pl.loop(0, NB0 // 2)
        def _(jj):
            j = 2 * jj
            pltpu.make_async_copy(table_hbm.at[src_v.at[j]], rows0,
                                  sem_g0).wait()

            @pl.when(jj > 0)
            def _():
                pltpu.make_async_copy(rows1, acc_sh.at[dst_v.at[j]],
                                      sem_s1).wait()

            pltpu.async_copy(table_hbm.at[src_v.at[j + 1]], rows1, sem_g1)
            pltpu.async_copy(rows0, acc_sh.at[dst_v.at[j]], sem_s0, add=True)
            pltpu.make_async_copy(table_hbm.at[src_v.at[j + 1]], rows1,
                                  sem_g1).wait()
            pltpu.make_async_copy(rows0, acc_sh.at[dst_v.at[j]],
                                  sem_s0).wait()

            @pl.when(jj < NB0 // 2 - 1)
            def _():
                pltpu.async_copy(table_hbm.at[src_v.at[j + 2]], rows0, sem_g0)

            pltpu.async_copy(rows1, acc_sh.at[dst_v.at[j + 1]], sem_s1,
                             add=True)

        pltpu.make_async_copy(rows1, acc_sh.at[dst_v.at[NB0 - 1]],
                              sem_s1).wait()

        @pl.when(w < NX)
        def _():
            pltpu.sync_copy(table_hbm.at[src_x.at[0]], rows0)
            pltpu.sync_copy(rows0, acc_sh.at[dst_x.at[0]], add=True)

        plsc.subcore_barrier()
        pltpu.sync_copy(acc_sh.at[pl.ds(r0, rows_per_sub)],
                        out_hbm.at[c, pl.ds(r0, rows_per_sub)])

    return edge_pass


_edge_pass_64 = _make_edge_pass(64)
_edge_pass_80 = _make_edge_pass(80)
_edge_pass_16 = _make_edge_pass(H)


def _mid0_body(a0_ref, a1_ref, b0_ref, b1_ref, x_ref, wl_ref, wr_ref, b_ref,
               h_ref, cm_ref):
    pA = a0_ref[...] + a1_ref[...]
    pB = b0_ref[...] + b1_ref[...]
    cm = jnp.maximum(pB[:, 64:], 1.0)
    p = jnp.concatenate([pA, pB[:, :64]], axis=1)
    agg = p / jnp.concatenate([cm] * (D_IN // H), axis=1)
    h_ref[...] = (jnp.dot(agg, wl_ref[...], preferred_element_type=_f32)
                  + b_ref[...]
                  + jnp.dot(x_ref[...], wr_ref[...],
                            preferred_element_type=_f32))
    cm_ref[...] = cm


_RB = 1000

_mid0 = pl.pallas_call(
    _mid0_body,
    grid=(N // _RB,),
    in_specs=[
        pl.BlockSpec((_RB, 64), lambda i: (i, 0)),
        pl.BlockSpec((_RB, 64), lambda i: (i, 0)),
        pl.BlockSpec((_RB, 80), lambda i: (i, 0)),
        pl.BlockSpec((_RB, 80), lambda i: (i, 0)),
        pl.BlockSpec((_RB, D_IN), lambda i: (i, 0)),
        pl.BlockSpec((D_IN, H), lambda i: (0, 0)),
        pl.BlockSpec((D_IN, H), lambda i: (0, 0)),
        pl.BlockSpec((1, H), lambda i: (0, 0)),
    ],
    out_specs=(pl.BlockSpec((_RB, H), lambda i: (i, 0)),
               pl.BlockSpec((_RB, H), lambda i: (i, 0))),
    out_shape=(jax.ShapeDtypeStruct((N, H), _f32),
               jax.ShapeDtypeStruct((N, H), _f32)),
)


def _mid_body(p0_ref, p1_ref, cm_ref, h_ref, wl_ref, wr_ref, b_ref, hn_ref):
    agg = (p0_ref[...] + p1_ref[...]) / cm_ref[...]
    hn_ref[...] = (jnp.dot(agg, wl_ref[...], preferred_element_type=_f32)
                   + b_ref[...]
                   + jnp.dot(h_ref[...], wr_ref[...],
                             preferred_element_type=_f32))


_mid = pl.pallas_call(
    _mid_body,
    out_shape=jax.ShapeDtypeStruct((N, H), _f32),
)


def _final_body(h_ref, wlin_ref, blin_ref, o_ref):
    pooled = jnp.sum(h_ref[...], axis=0, keepdims=True) / N
    o_ref[...] = jnp.dot(pooled, wlin_ref[...], preferred_element_type=_f32) \
        + blin_ref[...]


_final = pl.pallas_call(
    _final_body,
    out_shape=jax.ShapeDtypeStruct((1, 1), _f32),
)


def kernel(x, edge_index, Wl0, Wr0, b0, Wl1, Wr1, b1, Wl2, Wr2, b2, Wlin, blin):
    edge3 = edge_index.reshape(2, NB_TOT, CHUNK)
    zero64 = jnp.zeros((N, 64), _f32)
    zero80 = jnp.zeros((N, 80), _f32)
    zero16 = jnp.zeros((N, H), _f32)
    tableA = x[:, :64]
    tableB = jnp.concatenate([x[:, 64:], jnp.ones((N, H), _f32)], axis=1)

    paa = _edge_pass_64(tableA, edge3, zero64)
    pab = _edge_pass_80(tableB, edge3, zero80)
    h1, cm = _mid0(paa[0], paa[1], pab[0], pab[1], x, Wl0, Wr0,
                   b0.reshape(1, H))
    pb = _edge_pass_16(h1, edge3, zero16)
    h2 = _mid(pb[0], pb[1], cm, h1, Wl1, Wr1, b1.reshape(1, H))
    pc = _edge_pass_16(h2, edge3, zero16)
    h3 = _mid(pc[0], pc[1], cm, h2, Wl2, Wr2, b2.reshape(1, H))
    return _final(h3, Wlin, blin.reshape(1, 1))
